# baseline (speedup 1.0000x reference)
import jax
import jax.numpy as jnp
from jax import lax
from jax.experimental import pallas as pl
from jax.experimental.pallas import tpu as pltpu

N = 1024
F_IN = 128
HID = 128
HEADS = 4
OUT = 16
NG = 16
NCH = 12

TI1 = 256
NI1 = N // TI1
TI2 = 512
NI2 = N // TI2

_NEG = -1e30


def _fused_kernel(attn_ref, x_ref, wrep_ref, b_agg_ref, W1_ref,
                  asrc1_ref, adst1_ref, We1_ref, ae1_ref, b1_ref,
                  W2_ref, asrc2_ref, adst2_ref, We2_ref, ae2_ref, b2_ref,
                  batch_ref, fcw_ref, fcb_ref,
                  out_ref,
                  agg_scr, xs_scr, as_scr, ad_scr, acc_scr, s_scr, h1t_scr,
                  sel_scr):
    i = pl.program_id(0)

    @pl.when(i == 0)
    def _():
        xs = jnp.dot(x_ref[...], W1_ref[...],
                     preferred_element_type=jnp.float32)
        xs_scr[...] = xs
        for h in range(HEADS):
            c0 = h * HID
            xs_h = xs[:, c0:c0 + HID]
            as_scr[:, h:h + 1] = lax.dot_general(
                xs_h, asrc1_ref[h:h + 1, :], (((1,), (1,)), ((), ())),
                preferred_element_type=jnp.float32)
            ad_scr[h:h + 1, :] = lax.dot_general(
                adst1_ref[h:h + 1, :], xs_h, (((1,), (1,)), ((), ())),
                preferred_element_type=jnp.float32)
        acc_scr[...] = jnp.zeros_like(acc_scr)
        s_scr[...] = jnp.zeros_like(s_scr)
        r_iota = lax.broadcasted_iota(jnp.int32, (TI1, NCH * TI1), 0)
        c_iota = lax.broadcasted_iota(jnp.int32, (TI1, NCH * TI1), 1)
        sel_scr[...] = jnp.where(c_iota % TI1 == r_iota, wrep_ref[...], 0.0)

    a2 = attn_ref[...].reshape(NCH * TI1, N)
    agg = lax.dot_general(sel_scr[...], a2, (((1,), (0,)), ((), ())),
                          preferred_element_type=jnp.float32)
    agg = agg + b_agg_ref[0:1, 0:1]
    agg_scr[pl.ds(i * TI1, TI1), :] = agg
    mask = agg > 0.0

    for h in range(HEADS):
        c0 = h * HID
        xs_h = xs_scr[pl.ds(i * TI1, TI1), c0:c0 + HID]
        a_src = as_scr[pl.ds(i * TI1, TI1), h:h + 1]
        a_dst = ad_scr[h:h + 1, :]
        we_c = jnp.sum(We1_ref[0:1, c0:c0 + HID] * ae1_ref[h:h + 1, :],
                       axis=1, keepdims=True)
        alpha = a_src + a_dst + agg * we_c
        alpha = jnp.maximum(alpha, 0.2 * alpha)
        alpha = jnp.where(mask, alpha, _NEG)
        p = jnp.exp(alpha)
        s_scr[h:h + 1, :] = (s_scr[h:h + 1, :]
                             + jnp.sum(p, axis=0, keepdims=True))
        out_t = lax.dot_general(xs_h, p, (((0,), (0,)), ((), ())),
                                preferred_element_type=jnp.float32)
        acc_scr[c0:c0 + HID, :] = acc_scr[c0:c0 + HID, :] + out_t

    @pl.when(i == NI1 - 1)
    def _():
        for h in range(HEADS):
            c0 = h * HID
            s = s_scr[h:h + 1, :]
            r = jnp.where(s == 0.0, 0.0, 1.0 / (s + 1e-16))
            h1t_scr[c0:c0 + HID, :] = jnp.maximum(
                acc_scr[c0:c0 + HID, :] * r + b1_ref[c0:c0 + HID, :], 0.0)

        xs2 = lax.dot_general(h1t_scr[...], W2_ref[...],
                              (((0,), (0,)), ((), ())),
                              preferred_element_type=jnp.float32)
        a_src2 = lax.dot_general(xs2, asrc2_ref[...],
                                 (((1,), (1,)), ((), ())),
                                 preferred_element_type=jnp.float32)
        a_dst2 = lax.dot_general(adst2_ref[...], xs2,
                                 (((1,), (1,)), ((), ())),
                                 preferred_element_type=jnp.float32)
        we_c2 = jnp.sum(We2_ref[...] * ae2_ref[...], axis=1, keepdims=True)
        acc2 = jnp.zeros((HID, N), jnp.float32)
        s2 = jnp.zeros((1, N), jnp.float32)
        for c in range(NI2):
            r0 = c * TI2
            agg_c = agg_scr[r0:r0 + TI2, :]
            alpha2 = a_src2[r0:r0 + TI2, :] + a_dst2 + agg_c * we_c2
            alpha2 = jnp.maximum(alpha2, 0.2 * alpha2)
            alpha2 = jnp.where(agg_c > 0.0, alpha2, _NEG)
            p2 = jnp.exp(alpha2)
            s2 = s2 + jnp.sum(p2, axis=0, keepdims=True)
            acc2 = acc2 + lax.dot_general(
                xs2[r0:r0 + TI2, :], p2, (((0,), (0,)), ((), ())),
                preferred_element_type=jnp.float32)
        r2 = jnp.where(s2 == 0.0, 0.0, 1.0 / (s2 + 1e-16))
        h2_t = acc2 * r2 + b2_ref[...]

        groups = lax.broadcasted_iota(jnp.int32, (NG, N), 0)
        onehot = jnp.where(groups == batch_ref[...], 1.0, 0.0)
        sums = lax.dot_general(onehot, h2_t, (((1,), (1,)), ((), ())),
                               preferred_element_type=jnp.float32)
        counts = jnp.sum(onehot, axis=1, keepdims=True)
        pooled = sums / jnp.maximum(counts, 1.0)
        logits = jnp.dot(pooled, fcw_ref[...],
                         preferred_element_type=jnp.float32) + fcb_ref[...]
        mx = jnp.max(logits, axis=1, keepdims=True)
        z = logits - mx
        lse = jnp.log(jnp.sum(jnp.exp(z), axis=1, keepdims=True))
        out_ref[...] = z - lse


def kernel(x, batch_idx, attn_tensor, w_agg, b_agg,
           W1, att_src1, att_dst1, We1, att_e1, b1,
           W2, att_src2, att_dst2, We2, att_e2, b2,
           fc_w, fc_b):
    w_rep = jnp.reshape(
        jnp.repeat(w_agg.astype(jnp.float32), TI1), (1, NCH * TI1))
    b_agg2 = jnp.reshape(b_agg.astype(jnp.float32), (1, 1))
    b1_col = jnp.reshape(b1, (HEADS * HID, 1))
    b2_col = jnp.reshape(b2, (HID, 1))
    fcb2 = jnp.reshape(fc_b, (1, OUT))
    batch2 = jnp.reshape(batch_idx.astype(jnp.int32), (1, N))

    full = lambda shape: pl.BlockSpec(shape, lambda i: (0,) * len(shape))

    out = pl.pallas_call(
        _fused_kernel,
        grid=(NI1,),
        in_specs=[
            pl.BlockSpec((NCH, TI1, N), lambda i: (0, i, 0)),
            full((N, F_IN)),
            full((1, NCH * TI1)),
            full((1, 1)),
            full((F_IN, HEADS * HID)),
            full((HEADS, HID)),
            full((HEADS, HID)),
            full((1, HEADS * HID)),
            full((HEADS, HID)),
            full((HEADS * HID, 1)),
            full((HEADS * HID, HID)),
            full((1, HID)),
            full((1, HID)),
            full((1, HID)),
            full((1, HID)),
            full((HID, 1)),
            full((1, N)),
            full((HID, OUT)),
            full((1, OUT)),
        ],
        out_specs=pl.BlockSpec((NG, OUT), lambda i: (0, 0)),
        out_shape=jax.ShapeDtypeStruct((NG, OUT), jnp.float32),
        scratch_shapes=[
            pltpu.VMEM((N, N), jnp.float32),
            pltpu.VMEM((N, HEADS * HID), jnp.float32),
            pltpu.VMEM((N, 8), jnp.float32),
            pltpu.VMEM((8, N), jnp.float32),
            pltpu.VMEM((HEADS * HID, N), jnp.float32),
            pltpu.VMEM((8, N), jnp.float32),
            pltpu.VMEM((HEADS * HID, N), jnp.float32),
            pltpu.VMEM((TI1, NCH * TI1), jnp.float32),
        ],
        compiler_params=pltpu.CompilerParams(
            dimension_semantics=("arbitrary",)),
    )(attn_tensor, x, w_rep, b_agg2, W1, att_src1, att_dst1, We1,
      att_e1, b1_col, W2, att_src2, att_dst2, We2, att_e2, b2_col,
      batch2, fc_w, fcb2)
    return out


# speedup vs baseline: 1.0657x; 1.0657x over previous
import jax
import jax.numpy as jnp
from jax import lax
from jax.experimental import pallas as pl
from jax.experimental.pallas import tpu as pltpu

N = 1024
F_IN = 128
HID = 128
HEADS = 4
OUT = 16
NG = 16
NCH = 12

TI1 = 256
NI1 = N // TI1
TI2 = 512
NI2 = N // TI2

_NEG = -1e30


def _fused_kernel(attn_ref, x_ref, w_agg_ref, b_agg_ref, W1_ref,
                  asrc1_ref, adst1_ref, We1_ref, ae1_ref, b1_ref,
                  W2_ref, asrc2_ref, adst2_ref, We2_ref, ae2_ref, b2_ref,
                  batch_ref, fcw_ref, fcb_ref,
                  out_ref,
                  agg_scr, xs_scr, as_scr, ad_scr, acc_scr, s_scr, h1t_scr):
    i = pl.program_id(0)

    @pl.when(i == 0)
    def _():
        xs = jnp.dot(x_ref[...], W1_ref[...],
                     preferred_element_type=jnp.float32)
        xs_scr[...] = xs
        for h in range(HEADS):
            c0 = h * HID
            xs_h = xs[:, c0:c0 + HID]
            as_scr[:, h:h + 1] = lax.dot_general(
                xs_h, asrc1_ref[h:h + 1, :], (((1,), (1,)), ((), ())),
                preferred_element_type=jnp.float32)
            ad_scr[h:h + 1, :] = lax.dot_general(
                adst1_ref[h:h + 1, :], xs_h, (((1,), (1,)), ((), ())),
                preferred_element_type=jnp.float32)
        acc_scr[...] = jnp.zeros_like(acc_scr)
        s_scr[...] = jnp.zeros_like(s_scr)

    acc0 = attn_ref[0] * w_agg_ref[0:1, 0:1]
    acc1 = attn_ref[1] * w_agg_ref[0:1, 1:2]
    for k in range(2, NCH, 2):
        acc0 = acc0 + attn_ref[k] * w_agg_ref[0:1, k:k + 1]
        acc1 = acc1 + attn_ref[k + 1] * w_agg_ref[0:1, k + 1:k + 2]
    agg = acc0 + acc1 + b_agg_ref[0:1, 0:1]
    agg_scr[pl.ds(i * TI1, TI1), :] = agg
    mask = agg > 0.0

    for h in range(HEADS):
        c0 = h * HID
        xs_h = xs_scr[pl.ds(i * TI1, TI1), c0:c0 + HID]
        a_src = as_scr[pl.ds(i * TI1, TI1), h:h + 1]
        a_dst = ad_scr[h:h + 1, :]
        we_c = jnp.sum(We1_ref[0:1, c0:c0 + HID] * ae1_ref[h:h + 1, :],
                       axis=1, keepdims=True)
        alpha = a_src + a_dst + agg * we_c
        alpha = jnp.maximum(alpha, 0.2 * alpha)
        alpha = jnp.where(mask, alpha, _NEG)
        p = jnp.exp(alpha)
        s_scr[h:h + 1, :] = (s_scr[h:h + 1, :]
                             + jnp.sum(p, axis=0, keepdims=True))
        out_t = lax.dot_general(xs_h, p, (((0,), (0,)), ((), ())),
                                preferred_element_type=jnp.float32)
        acc_scr[c0:c0 + HID, :] = acc_scr[c0:c0 + HID, :] + out_t

    @pl.when(i == NI1 - 1)
    def _():
        for h in range(HEADS):
            c0 = h * HID
            s = s_scr[h:h + 1, :]
            r = jnp.where(s == 0.0, 0.0, 1.0 / (s + 1e-16))
            h1t_scr[c0:c0 + HID, :] = jnp.maximum(
                acc_scr[c0:c0 + HID, :] * r + b1_ref[c0:c0 + HID, :], 0.0)

        xs2 = lax.dot_general(h1t_scr[...], W2_ref[...],
                              (((0,), (0,)), ((), ())),
                              preferred_element_type=jnp.float32)
        a_src2 = lax.dot_general(xs2, asrc2_ref[...],
                                 (((1,), (1,)), ((), ())),
                                 preferred_element_type=jnp.float32)
        a_dst2 = lax.dot_general(adst2_ref[...], xs2,
                                 (((1,), (1,)), ((), ())),
                                 preferred_element_type=jnp.float32)
        we_c2 = jnp.sum(We2_ref[...] * ae2_ref[...], axis=1, keepdims=True)
        acc2 = jnp.zeros((HID, N), jnp.float32)
        s2 = jnp.zeros((1, N), jnp.float32)
        for c in range(NI2):
            r0 = c * TI2
            agg_c = agg_scr[r0:r0 + TI2, :]
            alpha2 = a_src2[r0:r0 + TI2, :] + a_dst2 + agg_c * we_c2
            alpha2 = jnp.maximum(alpha2, 0.2 * alpha2)
            alpha2 = jnp.where(agg_c > 0.0, alpha2, _NEG)
            p2 = jnp.exp(alpha2)
            s2 = s2 + jnp.sum(p2, axis=0, keepdims=True)
            acc2 = acc2 + lax.dot_general(
                xs2[r0:r0 + TI2, :], p2, (((0,), (0,)), ((), ())),
                preferred_element_type=jnp.float32)
        r2 = jnp.where(s2 == 0.0, 0.0, 1.0 / (s2 + 1e-16))
        h2_t = acc2 * r2 + b2_ref[...]

        groups = lax.broadcasted_iota(jnp.int32, (NG, N), 0)
        onehot = jnp.where(groups == batch_ref[...], 1.0, 0.0)
        sums = lax.dot_general(onehot, h2_t, (((1,), (1,)), ((), ())),
                               preferred_element_type=jnp.float32)
        counts = jnp.sum(onehot, axis=1, keepdims=True)
        pooled = sums / jnp.maximum(counts, 1.0)
        logits = jnp.dot(pooled, fcw_ref[...],
                         preferred_element_type=jnp.float32) + fcb_ref[...]
        mx = jnp.max(logits, axis=1, keepdims=True)
        z = logits - mx
        lse = jnp.log(jnp.sum(jnp.exp(z), axis=1, keepdims=True))
        out_ref[...] = z - lse


def kernel(x, batch_idx, attn_tensor, w_agg, b_agg,
           W1, att_src1, att_dst1, We1, att_e1, b1,
           W2, att_src2, att_dst2, We2, att_e2, b2,
           fc_w, fc_b):
    w_agg2 = jnp.reshape(w_agg.astype(jnp.float32), (1, NCH))
    b_agg2 = jnp.reshape(b_agg.astype(jnp.float32), (1, 1))
    b1_col = jnp.reshape(b1, (HEADS * HID, 1))
    b2_col = jnp.reshape(b2, (HID, 1))
    fcb2 = jnp.reshape(fc_b, (1, OUT))
    batch2 = jnp.reshape(batch_idx.astype(jnp.int32), (1, N))

    full = lambda shape: pl.BlockSpec(shape, lambda i: (0,) * len(shape))

    out = pl.pallas_call(
        _fused_kernel,
        grid=(NI1,),
        in_specs=[
            pl.BlockSpec((NCH, TI1, N), lambda i: (0, i, 0)),
            full((N, F_IN)),
            full((1, NCH)),
            full((1, 1)),
            full((F_IN, HEADS * HID)),
            full((HEADS, HID)),
            full((HEADS, HID)),
            full((1, HEADS * HID)),
            full((HEADS, HID)),
            full((HEADS * HID, 1)),
            full((HEADS * HID, HID)),
            full((1, HID)),
            full((1, HID)),
            full((1, HID)),
            full((1, HID)),
            full((HID, 1)),
            full((1, N)),
            full((HID, OUT)),
            full((1, OUT)),
        ],
        out_specs=pl.BlockSpec((NG, OUT), lambda i: (0, 0)),
        out_shape=jax.ShapeDtypeStruct((NG, OUT), jnp.float32),
        scratch_shapes=[
            pltpu.VMEM((N, N), jnp.float32),
            pltpu.VMEM((N, HEADS * HID), jnp.float32),
            pltpu.VMEM((N, 8), jnp.float32),
            pltpu.VMEM((8, N), jnp.float32),
            pltpu.VMEM((HEADS * HID, N), jnp.float32),
            pltpu.VMEM((8, N), jnp.float32),
            pltpu.VMEM((HEADS * HID, N), jnp.float32),
        ],
        compiler_params=pltpu.CompilerParams(
            dimension_semantics=("arbitrary",)),
    )(attn_tensor, x, w_agg2, b_agg2, W1, att_src1, att_dst1, We1,
      att_e1, b1_col, W2, att_src2, att_dst2, We2, att_e2, b2_col,
      batch2, fc_w, fcb2)
    return out


# speedup vs baseline: 1.1920x; 1.1184x over previous
import jax
import jax.numpy as jnp
from jax import lax
from jax.experimental import pallas as pl
from jax.experimental.pallas import tpu as pltpu

N = 1024
F_IN = 128
HID = 128
HEADS = 4
OUT = 16
NG = 16
NCH = 12

TI1 = 256
NI1 = N // TI1
TI2 = 512
NI2 = N // TI2

_NEG = -1e30


def _fused_kernel(attn_ref, x_ref, w_agg_ref, b_agg_ref, W1_ref,
                  asrc1_ref, adst1_ref, We1_ref, ae1_ref, b1_ref,
                  W2_ref, asrc2_ref, adst2_ref, We2_ref, ae2_ref, b2_ref,
                  batch_ref, fcw_ref, fcb_ref,
                  out_ref,
                  agg_scr, xs_scr, as_scr, ad_scr, acc_scr, s_scr, h1t_scr,
                  b_scr):
    i = pl.program_id(0)

    @pl.when(i == 0)
    def _():
        b_scr[:, 0:1] = jnp.transpose(b1_ref[...], (1, 0))
        b_scr[0:HID, 1:2] = jnp.transpose(b2_ref[...], (1, 0))
        xs = jnp.dot(x_ref[...], W1_ref[...],
                     preferred_element_type=jnp.float32)
        xs_scr[...] = xs
        for h in range(HEADS):
            c0 = h * HID
            xs_h = xs[:, c0:c0 + HID]
            as_scr[:, h:h + 1] = lax.dot_general(
                xs_h, asrc1_ref[h:h + 1, :], (((1,), (1,)), ((), ())),
                preferred_element_type=jnp.float32)
            ad_scr[h:h + 1, :] = lax.dot_general(
                adst1_ref[h:h + 1, :], xs_h, (((1,), (1,)), ((), ())),
                preferred_element_type=jnp.float32)
        acc_scr[...] = jnp.zeros_like(acc_scr)
        s_scr[...] = jnp.zeros_like(s_scr)

    acc0 = attn_ref[0] * w_agg_ref[0:1, 0:1]
    acc1 = attn_ref[1] * w_agg_ref[0:1, 1:2]
    for k in range(2, NCH, 2):
        acc0 = acc0 + attn_ref[k] * w_agg_ref[0:1, k:k + 1]
        acc1 = acc1 + attn_ref[k + 1] * w_agg_ref[0:1, k + 1:k + 2]
    agg = acc0 + acc1 + b_agg_ref[0:1, 0:1]
    agg_scr[pl.ds(i * TI1, TI1), :] = agg
    mask = agg > 0.0

    for h in range(HEADS):
        c0 = h * HID
        xs_h = xs_scr[pl.ds(i * TI1, TI1), c0:c0 + HID]
        a_src = as_scr[pl.ds(i * TI1, TI1), h:h + 1]
        a_dst = ad_scr[h:h + 1, :]
        we_c = jnp.sum(We1_ref[0:1, c0:c0 + HID] * ae1_ref[h:h + 1, :],
                       axis=1, keepdims=True)
        alpha = a_src + a_dst + agg * we_c
        alpha = jnp.maximum(alpha, 0.2 * alpha)
        alpha = jnp.where(mask, alpha, _NEG)
        p = jnp.exp(alpha)
        s_scr[h:h + 1, :] = (s_scr[h:h + 1, :]
                             + jnp.sum(p, axis=0, keepdims=True))
        out_t = lax.dot_general(xs_h, p, (((0,), (0,)), ((), ())),
                                preferred_element_type=jnp.float32)
        acc_scr[c0:c0 + HID, :] = acc_scr[c0:c0 + HID, :] + out_t

    @pl.when(i == NI1 - 1)
    def _():
        for h in range(HEADS):
            c0 = h * HID
            s = s_scr[h:h + 1, :]
            r = jnp.where(s == 0.0, 0.0, 1.0 / (s + 1e-16))
            h1t_scr[c0:c0 + HID, :] = jnp.maximum(
                acc_scr[c0:c0 + HID, :] * r + b_scr[c0:c0 + HID, 0:1], 0.0)

        xs2 = lax.dot_general(h1t_scr[...], W2_ref[...],
                              (((0,), (0,)), ((), ())),
                              preferred_element_type=jnp.float32)
        a_src2 = lax.dot_general(xs2, asrc2_ref[...],
                                 (((1,), (1,)), ((), ())),
                                 preferred_element_type=jnp.float32)
        a_dst2 = lax.dot_general(adst2_ref[...], xs2,
                                 (((1,), (1,)), ((), ())),
                                 preferred_element_type=jnp.float32)
        we_c2 = jnp.sum(We2_ref[...] * ae2_ref[...], axis=1, keepdims=True)
        acc2 = jnp.zeros((HID, N), jnp.float32)
        s2 = jnp.zeros((1, N), jnp.float32)
        for c in range(NI2):
            r0 = c * TI2
            agg_c = agg_scr[r0:r0 + TI2, :]
            alpha2 = a_src2[r0:r0 + TI2, :] + a_dst2 + agg_c * we_c2
            alpha2 = jnp.maximum(alpha2, 0.2 * alpha2)
            alpha2 = jnp.where(agg_c > 0.0, alpha2, _NEG)
            p2 = jnp.exp(alpha2)
            s2 = s2 + jnp.sum(p2, axis=0, keepdims=True)
            acc2 = acc2 + lax.dot_general(
                xs2[r0:r0 + TI2, :], p2, (((0,), (0,)), ((), ())),
                preferred_element_type=jnp.float32)
        r2 = jnp.where(s2 == 0.0, 0.0, 1.0 / (s2 + 1e-16))
        h2_t = acc2 * r2 + b_scr[0:HID, 1:2]

        groups = lax.broadcasted_iota(jnp.int32, (NG, N), 0)
        onehot = jnp.where(groups == batch_ref[...], 1.0, 0.0)
        sums = lax.dot_general(onehot, h2_t, (((1,), (1,)), ((), ())),
                               preferred_element_type=jnp.float32)
        counts = jnp.sum(onehot, axis=1, keepdims=True)
        pooled = sums / jnp.maximum(counts, 1.0)
        logits = jnp.dot(pooled, fcw_ref[...],
                         preferred_element_type=jnp.float32) + fcb_ref[...]
        mx = jnp.max(logits, axis=1, keepdims=True)
        z = logits - mx
        lse = jnp.log(jnp.sum(jnp.exp(z), axis=1, keepdims=True))
        out_ref[...] = z - lse


def kernel(x, batch_idx, attn_tensor, w_agg, b_agg,
           W1, att_src1, att_dst1, We1, att_e1, b1,
           W2, att_src2, att_dst2, We2, att_e2, b2,
           fc_w, fc_b):
    w_agg2 = jnp.reshape(w_agg.astype(jnp.float32), (1, NCH))
    b_agg2 = jnp.reshape(b_agg.astype(jnp.float32), (1, 1))
    b1_row = jnp.reshape(b1, (1, HEADS * HID))
    b2_row = jnp.reshape(b2, (1, HID))
    fcb2 = jnp.reshape(fc_b, (1, OUT))
    batch2 = jnp.reshape(batch_idx.astype(jnp.int32), (1, N))

    full = lambda shape: pl.BlockSpec(shape, lambda i: (0,) * len(shape))

    out = pl.pallas_call(
        _fused_kernel,
        grid=(NI1,),
        in_specs=[
            pl.BlockSpec((NCH, TI1, N), lambda i: (0, i, 0)),
            full((N, F_IN)),
            full((1, NCH)),
            full((1, 1)),
            full((F_IN, HEADS * HID)),
            full((HEADS, HID)),
            full((HEADS, HID)),
            full((1, HEADS * HID)),
            full((HEADS, HID)),
            full((1, HEADS * HID)),
            full((HEADS * HID, HID)),
            full((1, HID)),
            full((1, HID)),
            full((1, HID)),
            full((1, HID)),
            full((1, HID)),
            full((1, N)),
            full((HID, OUT)),
            full((1, OUT)),
        ],
        out_specs=pl.BlockSpec((NG, OUT), lambda i: (0, 0)),
        out_shape=jax.ShapeDtypeStruct((NG, OUT), jnp.float32),
        scratch_shapes=[
            pltpu.VMEM((N, N), jnp.float32),
            pltpu.VMEM((N, HEADS * HID), jnp.float32),
            pltpu.VMEM((N, 8), jnp.float32),
            pltpu.VMEM((8, N), jnp.float32),
            pltpu.VMEM((HEADS * HID, N), jnp.float32),
            pltpu.VMEM((8, N), jnp.float32),
            pltpu.VMEM((HEADS * HID, N), jnp.float32),
            pltpu.VMEM((HEADS * HID, 8), jnp.float32),
        ],
        compiler_params=pltpu.CompilerParams(
            dimension_semantics=("arbitrary",)),
    )(attn_tensor, x, w_agg2, b_agg2, W1, att_src1, att_dst1, We1,
      att_e1, b1_row, W2, att_src2, att_dst2, We2, att_e2, b2_row,
      batch2, fc_w, fcb2)
    return out


# speedup vs baseline: 1.2582x; 1.0556x over previous
import jax
import jax.numpy as jnp
from jax import lax
from jax.experimental import pallas as pl
from jax.experimental.pallas import tpu as pltpu

N = 1024
F_IN = 128
HID = 128
HEADS = 4
OUT = 16
NG = 16
NCH = 12

TI1 = 256
NI1 = N // TI1
TI2 = 512
NI2 = N // TI2

_NEG = -1e30


def _fused_kernel(attn_ref, x_ref, w_agg_ref, b_agg_ref, W1_ref,
                  asrc1_ref, adst1_ref, We1_ref, ae1_ref, b1_ref,
                  W2_ref, asrc2_ref, adst2_ref, We2_ref, ae2_ref, b2_ref,
                  batch_ref, fcw_ref, fcb_ref,
                  out_ref,
                  agg_scr, xs_scr, as_scr, ad_scr, acc_scr, s_scr, h1t_scr,
                  b_scr):
    i = pl.program_id(0)

    @pl.when(i == 0)
    def _():
        b_scr[:, 0:1] = jnp.transpose(b1_ref[...], (1, 0))
        b_scr[0:HID, 1:2] = jnp.transpose(b2_ref[...], (1, 0))
        xs = jnp.dot(x_ref[...], W1_ref[...],
                     preferred_element_type=jnp.float32)
        xs_scr[...] = xs
        for h in range(HEADS):
            c0 = h * HID
            xs_h = xs[:, c0:c0 + HID]
            as_scr[:, h:h + 1] = lax.dot_general(
                xs_h, asrc1_ref[h:h + 1, :], (((1,), (1,)), ((), ())),
                preferred_element_type=jnp.float32)
            ad_scr[h:h + 1, :] = lax.dot_general(
                adst1_ref[h:h + 1, :], xs_h, (((1,), (1,)), ((), ())),
                preferred_element_type=jnp.float32)
        acc_scr[...] = jnp.zeros_like(acc_scr)
        s_scr[...] = jnp.zeros_like(s_scr)

    acc0 = attn_ref[0] * w_agg_ref[0:1, 0:1]
    acc1 = attn_ref[1] * w_agg_ref[0:1, 1:2]
    for k in range(2, NCH, 2):
        acc0 = acc0 + attn_ref[k] * w_agg_ref[0:1, k:k + 1]
        acc1 = acc1 + attn_ref[k + 1] * w_agg_ref[0:1, k + 1:k + 2]
    agg = acc0 + acc1 + b_agg_ref[0:1, 0:1]
    agg_scr[pl.ds(i * TI1, TI1), :] = agg
    mask = agg > 0.0

    for h in range(HEADS):
        c0 = h * HID
        xs_h = xs_scr[pl.ds(i * TI1, TI1), c0:c0 + HID]
        a_src = as_scr[pl.ds(i * TI1, TI1), h:h + 1]
        a_dst = ad_scr[h:h + 1, :]
        we_c = jnp.sum(We1_ref[0:1, c0:c0 + HID] * ae1_ref[h:h + 1, :],
                       axis=1, keepdims=True)
        alpha = a_src + a_dst + agg * we_c
        alpha = jnp.maximum(alpha, 0.2 * alpha)
        alpha = jnp.where(mask, alpha, _NEG)
        p = jnp.exp(alpha)
        s_scr[h:h + 1, :] = (s_scr[h:h + 1, :]
                             + jnp.sum(p, axis=0, keepdims=True))
        out_t = lax.dot_general(xs_h, p, (((0,), (0,)), ((), ())),
                                preferred_element_type=jnp.float32)
        acc_scr[c0:c0 + HID, :] = acc_scr[c0:c0 + HID, :] + out_t

    @pl.when(i == NI1 - 1)
    def _():
        for h in range(HEADS):
            c0 = h * HID
            s = s_scr[h:h + 1, :]
            r = jnp.where(s == 0.0, 0.0, 1.0 / (s + 1e-16))
            h1t_scr[c0:c0 + HID, :] = jnp.maximum(
                acc_scr[c0:c0 + HID, :] * r + b_scr[c0:c0 + HID, 0:1], 0.0)

        xs2 = lax.dot_general(h1t_scr[...], W2_ref[...],
                              (((0,), (0,)), ((), ())),
                              preferred_element_type=jnp.float32)
        a_src2 = lax.dot_general(xs2, asrc2_ref[...],
                                 (((1,), (1,)), ((), ())),
                                 preferred_element_type=jnp.float32)
        a_dst2 = lax.dot_general(adst2_ref[...], xs2,
                                 (((1,), (1,)), ((), ())),
                                 preferred_element_type=jnp.float32)
        we_c2 = jnp.sum(We2_ref[...] * ae2_ref[...], axis=1, keepdims=True)
        acc2 = jnp.zeros((HID, N), jnp.float32)
        s2 = jnp.zeros((1, N), jnp.float32)
        for c in range(NI2):
            r0 = c * TI2
            agg_c = agg_scr[r0:r0 + TI2, :]
            alpha2 = a_src2[r0:r0 + TI2, :] + a_dst2 + agg_c * we_c2
            alpha2 = jnp.maximum(alpha2, 0.2 * alpha2)
            alpha2 = jnp.where(agg_c > 0.0, alpha2, _NEG)
            p2 = jnp.exp(alpha2)
            s2 = s2 + jnp.sum(p2, axis=0, keepdims=True)
            acc2 = acc2 + lax.dot_general(
                xs2[r0:r0 + TI2, :], p2, (((0,), (0,)), ((), ())),
                preferred_element_type=jnp.float32)
        r2 = jnp.where(s2 == 0.0, 0.0, 1.0 / (s2 + 1e-16))
        h2_t = acc2 * r2 + b_scr[0:HID, 1:2]

        groups = lax.broadcasted_iota(jnp.int32, (NG, N), 0)
        onehot = jnp.where(groups == batch_ref[...], 1.0, 0.0)
        sums = lax.dot_general(onehot, h2_t, (((1,), (1,)), ((), ())),
                               preferred_element_type=jnp.float32)
        counts = jnp.sum(onehot, axis=1, keepdims=True)
        pooled = sums / jnp.maximum(counts, 1.0)
        logits = lax.dot_general(pooled, fcw_ref[...],
                                 (((1,), (1,)), ((), ())),
                                 preferred_element_type=jnp.float32)
        logits = logits + fcb_ref[...]
        mx = jnp.max(logits, axis=1, keepdims=True)
        z = logits - mx
        lse = jnp.log(jnp.sum(jnp.exp(z), axis=1, keepdims=True))
        out_ref[...] = z - lse


def kernel(x, batch_idx, attn_tensor, w_agg, b_agg,
           W1, att_src1, att_dst1, We1, att_e1, b1,
           W2, att_src2, att_dst2, We2, att_e2, b2,
           fc_w, fc_b):
    w_agg2 = jnp.reshape(w_agg.astype(jnp.float32), (1, NCH))
    b_agg2 = jnp.reshape(b_agg.astype(jnp.float32), (1, 1))
    b1_row = jnp.reshape(b1, (1, HEADS * HID))
    b2_row = jnp.reshape(b2, (1, HID))
    fcb2 = jnp.reshape(fc_b, (1, OUT))
    fcw_t = jnp.transpose(fc_w)
    batch2 = jnp.reshape(batch_idx.astype(jnp.int32), (1, N))

    full = lambda shape: pl.BlockSpec(shape, lambda i: (0,) * len(shape))

    out = pl.pallas_call(
        _fused_kernel,
        grid=(NI1,),
        in_specs=[
            pl.BlockSpec((NCH, TI1, N), lambda i: (0, i, 0)),
            full((N, F_IN)),
            full((1, NCH)),
            full((1, 1)),
            full((F_IN, HEADS * HID)),
            full((HEADS, HID)),
            full((HEADS, HID)),
            full((1, HEADS * HID)),
            full((HEADS, HID)),
            full((1, HEADS * HID)),
            full((HEADS * HID, HID)),
            full((1, HID)),
            full((1, HID)),
            full((1, HID)),
            full((1, HID)),
            full((1, HID)),
            full((1, N)),
            full((OUT, HID)),
            full((1, OUT)),
        ],
        out_specs=pl.BlockSpec((NG, OUT), lambda i: (0, 0)),
        out_shape=jax.ShapeDtypeStruct((NG, OUT), jnp.float32),
        scratch_shapes=[
            pltpu.VMEM((N, N), jnp.float32),
            pltpu.VMEM((N, HEADS * HID), jnp.float32),
            pltpu.VMEM((N, 8), jnp.float32),
            pltpu.VMEM((8, N), jnp.float32),
            pltpu.VMEM((HEADS * HID, N), jnp.float32),
            pltpu.VMEM((8, N), jnp.float32),
            pltpu.VMEM((HEADS * HID, N), jnp.float32),
            pltpu.VMEM((HEADS * HID, 8), jnp.float32),
        ],
        compiler_params=pltpu.CompilerParams(
            dimension_semantics=("arbitrary",)),
    )(attn_tensor, x, w_agg2, b_agg2, W1, att_src1, att_dst1, We1,
      att_e1, b1_row, W2, att_src2, att_dst2, We2, att_e2, b2_row,
      batch2, fcw_t, fcb2)
    return out


# speedup vs baseline: 1.2668x; 1.0068x over previous
import jax
import jax.numpy as jnp
from jax import lax
from jax.experimental import pallas as pl
from jax.experimental.pallas import tpu as pltpu

N = 1024
F_IN = 128
HID = 128
HEADS = 4
OUT = 16
NG = 16
NCH = 12

TI1 = 256
NI1 = N // TI1
TI2 = 512
NI2 = N // TI2

_NEG = -1e30


def _fused_kernel(attn_ref, x_ref, w_agg_ref, b_agg_ref, W1_ref,
                  asrc1_ref, adst1_ref, We1_ref, ae1_ref, b1_ref,
                  W2_ref, asrc2_ref, adst2_ref, We2_ref, ae2_ref, b2_ref,
                  batch_ref, fcw_ref, fcb_ref,
                  out_ref,
                  agg_scr, xs_scr, as_scr, ad_scr, acc_scr, s_scr, h1t_scr,
                  b_scr, fcw_scr):
    i = pl.program_id(0)

    @pl.when(i == 0)
    def _():
        b_scr[:, 0:1] = jnp.transpose(b1_ref[...], (1, 0))
        b_scr[0:HID, 1:2] = jnp.transpose(b2_ref[...], (1, 0))
        fcw_scr[...] = jnp.transpose(fcw_ref[...], (1, 0))
        xs = jnp.dot(x_ref[...], W1_ref[...],
                     preferred_element_type=jnp.float32)
        xs_scr[...] = xs
        for h in range(HEADS):
            c0 = h * HID
            xs_h = xs[:, c0:c0 + HID]
            as_scr[:, h:h + 1] = lax.dot_general(
                xs_h, asrc1_ref[h:h + 1, :], (((1,), (1,)), ((), ())),
                preferred_element_type=jnp.float32)
            ad_scr[h:h + 1, :] = lax.dot_general(
                adst1_ref[h:h + 1, :], xs_h, (((1,), (1,)), ((), ())),
                preferred_element_type=jnp.float32)
        acc_scr[...] = jnp.zeros_like(acc_scr)
        s_scr[...] = jnp.zeros_like(s_scr)

    acc0 = attn_ref[0] * w_agg_ref[0:1, 0:1]
    acc1 = attn_ref[1] * w_agg_ref[0:1, 1:2]
    for k in range(2, NCH, 2):
        acc0 = acc0 + attn_ref[k] * w_agg_ref[0:1, k:k + 1]
        acc1 = acc1 + attn_ref[k + 1] * w_agg_ref[0:1, k + 1:k + 2]
    agg = acc0 + acc1 + b_agg_ref[0:1, 0:1]
    agg_scr[pl.ds(i * TI1, TI1), :] = agg
    mask = agg > 0.0

    for h in range(HEADS):
        c0 = h * HID
        xs_h = xs_scr[pl.ds(i * TI1, TI1), c0:c0 + HID]
        a_src = as_scr[pl.ds(i * TI1, TI1), h:h + 1]
        a_dst = ad_scr[h:h + 1, :]
        we_c = jnp.sum(We1_ref[0:1, c0:c0 + HID] * ae1_ref[h:h + 1, :],
                       axis=1, keepdims=True)
        alpha = a_src + a_dst + agg * we_c
        alpha = jnp.maximum(alpha, 0.2 * alpha)
        alpha = jnp.where(mask, alpha, _NEG)
        p = jnp.exp(alpha)
        s_scr[h:h + 1, :] = (s_scr[h:h + 1, :]
                             + jnp.sum(p, axis=0, keepdims=True))
        out_t = lax.dot_general(xs_h, p, (((0,), (0,)), ((), ())),
                                preferred_element_type=jnp.float32)
        acc_scr[c0:c0 + HID, :] = acc_scr[c0:c0 + HID, :] + out_t

    @pl.when(i == NI1 - 1)
    def _():
        for h in range(HEADS):
            c0 = h * HID
            s = s_scr[h:h + 1, :]
            r = jnp.where(s == 0.0, 0.0, 1.0 / (s + 1e-16))
            h1t_scr[c0:c0 + HID, :] = jnp.maximum(
                acc_scr[c0:c0 + HID, :] * r + b_scr[c0:c0 + HID, 0:1], 0.0)

        xs2 = lax.dot_general(h1t_scr[...], W2_ref[...],
                              (((0,), (0,)), ((), ())),
                              preferred_element_type=jnp.float32)
        a_src2 = lax.dot_general(xs2, asrc2_ref[...],
                                 (((1,), (1,)), ((), ())),
                                 preferred_element_type=jnp.float32)
        a_dst2 = lax.dot_general(adst2_ref[...], xs2,
                                 (((1,), (1,)), ((), ())),
                                 preferred_element_type=jnp.float32)
        we_c2 = jnp.sum(We2_ref[...] * ae2_ref[...], axis=1, keepdims=True)
        acc2 = jnp.zeros((HID, N), jnp.float32)
        s2 = jnp.zeros((1, N), jnp.float32)
        for c in range(NI2):
            r0 = c * TI2
            agg_c = agg_scr[r0:r0 + TI2, :]
            alpha2 = a_src2[r0:r0 + TI2, :] + a_dst2 + agg_c * we_c2
            alpha2 = jnp.maximum(alpha2, 0.2 * alpha2)
            alpha2 = jnp.where(agg_c > 0.0, alpha2, _NEG)
            p2 = jnp.exp(alpha2)
            s2 = s2 + jnp.sum(p2, axis=0, keepdims=True)
            acc2 = acc2 + lax.dot_general(
                xs2[r0:r0 + TI2, :], p2, (((0,), (0,)), ((), ())),
                preferred_element_type=jnp.float32)
        r2 = jnp.where(s2 == 0.0, 0.0, 1.0 / (s2 + 1e-16))
        h2_t = acc2 * r2 + b_scr[0:HID, 1:2]

        groups = lax.broadcasted_iota(jnp.int32, (NG, N), 0)
        onehot = jnp.where(groups == batch_ref[...], 1.0, 0.0)
        sums = lax.dot_general(onehot, h2_t, (((1,), (1,)), ((), ())),
                               preferred_element_type=jnp.float32)
        counts = jnp.sum(onehot, axis=1, keepdims=True)
        pooled = sums / jnp.maximum(counts, 1.0)
        logits = jnp.dot(pooled, fcw_scr[...],
                         preferred_element_type=jnp.float32) + fcb_ref[...]
        mx = jnp.max(logits, axis=1, keepdims=True)
        z = logits - mx
        lse = jnp.log(jnp.sum(jnp.exp(z), axis=1, keepdims=True))
        out_ref[...] = z - lse


def kernel(x, batch_idx, attn_tensor, w_agg, b_agg,
           W1, att_src1, att_dst1, We1, att_e1, b1,
           W2, att_src2, att_dst2, We2, att_e2, b2,
           fc_w, fc_b):
    w_agg2 = jnp.reshape(w_agg.astype(jnp.float32), (1, NCH))
    b_agg2 = jnp.reshape(b_agg.astype(jnp.float32), (1, 1))
    b1_row = jnp.reshape(b1, (1, HEADS * HID))
    b2_row = jnp.reshape(b2, (1, HID))
    fcb2 = jnp.reshape(fc_b, (1, OUT))
    fcw_t = jnp.transpose(fc_w)
    batch2 = jnp.reshape(batch_idx.astype(jnp.int32), (1, N))

    full = lambda shape: pl.BlockSpec(shape, lambda i: (0,) * len(shape))

    out = pl.pallas_call(
        _fused_kernel,
        grid=(NI1,),
        in_specs=[
            pl.BlockSpec((NCH, TI1, N), lambda i: (0, i, 0)),
            full((N, F_IN)),
            full((1, NCH)),
            full((1, 1)),
            full((F_IN, HEADS * HID)),
            full((HEADS, HID)),
            full((HEADS, HID)),
            full((1, HEADS * HID)),
            full((HEADS, HID)),
            full((1, HEADS * HID)),
            full((HEADS * HID, HID)),
            full((1, HID)),
            full((1, HID)),
            full((1, HID)),
            full((1, HID)),
            full((1, HID)),
            full((1, N)),
            full((OUT, HID)),
            full((1, OUT)),
        ],
        out_specs=pl.BlockSpec((NG, OUT), lambda i: (0, 0)),
        out_shape=jax.ShapeDtypeStruct((NG, OUT), jnp.float32),
        scratch_shapes=[
            pltpu.VMEM((N, N), jnp.float32),
            pltpu.VMEM((N, HEADS * HID), jnp.float32),
            pltpu.VMEM((N, 8), jnp.float32),
            pltpu.VMEM((8, N), jnp.float32),
            pltpu.VMEM((HEADS * HID, N), jnp.float32),
            pltpu.VMEM((8, N), jnp.float32),
            pltpu.VMEM((HEADS * HID, N), jnp.float32),
            pltpu.VMEM((HEADS * HID, 8), jnp.float32),
            pltpu.VMEM((HID, OUT), jnp.float32),
        ],
        compiler_params=pltpu.CompilerParams(
            dimension_semantics=("arbitrary",)),
    )(attn_tensor, x, w_agg2, b_agg2, W1, att_src1, att_dst1, We1,
      att_e1, b1_row, W2, att_src2, att_dst2, We2, att_e2, b2_row,
      batch2, fcw_t, fcb2)
    return out
